# speedup vs baseline: 1.4684x; 1.4684x over previous
import functools

import jax
import jax.numpy as jnp
from jax import lax
from jax.experimental import pallas as pl
from jax.experimental.pallas import tpu as pltpu
from jax.experimental.pallas import tpu_sc as plsc

D_MODEL = 1024
SCALE = 32.0

_NC = 2
_NS = 16
_NW = _NC * _NS

_B = 4 * 4096
_BPW = _B // _NW
_C = 32
_NCHUNK = _BPW // _C
_NBUF = 3
_LANES = 16
_SLICES_PER_ROW = D_MODEL // _LANES


def _emb_body(x_hbm, table_hbm, out_hbm, idx_v, rows_v,
              gsem0, gsem1, gsem2, ssem0, ssem1, ssem2):
    gsems = [gsem0, gsem1, gsem2]
    ssems = [ssem0, ssem1, ssem2]
    wid = lax.axis_index("s") * _NC + lax.axis_index("c")
    base = wid * _BPW

    pltpu.sync_copy(x_hbm.at[pl.ds(base, _BPW)], idx_v)

    def start_gather(c, b):
        return pltpu.async_copy(
            table_hbm.at[idx_v.at[pl.ds(c * _C, _C)]], rows_v.at[b], gsems[b])

    def start_scatter(c, b):
        return pltpu.async_copy(
            rows_v.at[b], out_hbm.at[pl.ds(base + c * _C, _C)], ssems[b])

    def scale_chunk(b):
        def row_body(r, carry):
            for j in range(_SLICES_PER_ROW):
                sl = pl.ds(j * _LANES, _LANES)
                rows_v[b, r, sl] = rows_v[b, r, sl] * SCALE
            return carry
        lax.fori_loop(0, _C, row_body, 0)

    gathers = [None] * _NCHUNK
    scatters = [None] * _NCHUNK
    for c in range(_NBUF - 1):
        gathers[c] = start_gather(c, c % _NBUF)

    for c in range(_NCHUNK):
        b = c % _NBUF
        gathers[c].wait()
        scale_chunk(b)
        scatters[c] = start_scatter(c, b)
        nc = c + _NBUF - 1
        if nc < _NCHUNK:
            if nc - _NBUF >= 0:
                scatters[nc - _NBUF].wait()
            gathers[nc] = start_gather(nc, nc % _NBUF)

    for c in range(_NCHUNK - _NBUF, _NCHUNK):
        scatters[c].wait()


@jax.jit
def kernel(x, table):
    xf = x.reshape(-1).astype(jnp.int32)

    mesh = plsc.VectorSubcoreMesh(core_axis_name="c", subcore_axis_name="s")
    run = functools.partial(
        pl.kernel,
        mesh=mesh,
        out_type=jax.ShapeDtypeStruct((_B, D_MODEL), jnp.float32),
        scratch_types=[
            pltpu.VMEM((_BPW,), jnp.int32),
            pltpu.VMEM((_NBUF, _C, D_MODEL), jnp.float32),
            pltpu.SemaphoreType.DMA,
            pltpu.SemaphoreType.DMA,
            pltpu.SemaphoreType.DMA,
            pltpu.SemaphoreType.DMA,
            pltpu.SemaphoreType.DMA,
            pltpu.SemaphoreType.DMA,
        ],
    )(_emb_body)
    out = run(xf, table)
    return out.reshape(x.shape + (D_MODEL,))
